# baseline (speedup 1.0000x reference)
import jax
import jax.numpy as jnp
from jax.experimental import pallas as pl
from jax.experimental.pallas import tpu as pltpu

_T = 8
_R = 16
_TR = _T * _R
_SCALING = 32.0 / 16.0
_MBLK = 256


def _fused_body(x_ref, tid_ref, w_ref, b_ref, a_ref, bcat_ref, o_ref, wb_ref):
    @pl.when(pl.program_id(0) == 0)
    def _():
        wb_ref[...] = w_ref[...].astype(jnp.bfloat16)

    x = x_ref[...].astype(jnp.bfloat16)
    base = jax.lax.dot_general(
        x, wb_ref[...], (((1,), (1,)), ((), ())),
        preferred_element_type=jnp.float32)
    z = jax.lax.dot_general(
        x, a_ref[...], (((1,), (1,)), ((), ())),
        preferred_element_type=jnp.float32)
    tid = tid_ref[...]
    col = jax.lax.broadcasted_iota(jnp.int32, z.shape, 1)
    z = jnp.where((col >> 4) == tid, z, 0.0).astype(jnp.bfloat16)
    lora = jax.lax.dot_general(
        z, bcat_ref[...], (((1,), (0,)), ((), ())),
        preferred_element_type=jnp.float32)
    o_ref[...] = base + (lora + b_ref[...])


def kernel(x, task_ids, W, b, lora_A, lora_B):
    ntok, din = x.shape
    dout = W.shape[0]
    tid2d = task_ids.astype(jnp.int32).reshape(ntok, 1)
    a_cat = lora_A.reshape(_TR, din).astype(jnp.bfloat16)
    b_cat = jnp.transpose(lora_B, (0, 2, 1)).reshape(_TR, dout)
    b_cat = (b_cat * _SCALING).astype(jnp.bfloat16)
    b2d = b.reshape(1, dout)

    grid = (ntok // _MBLK,)
    return pl.pallas_call(
        _fused_body,
        grid=grid,
        in_specs=[
            pl.BlockSpec((_MBLK, din), lambda i: (i, 0)),
            pl.BlockSpec((_MBLK, 1), lambda i: (i, 0)),
            pl.BlockSpec((dout, din), lambda i: (0, 0)),
            pl.BlockSpec((1, dout), lambda i: (0, 0)),
            pl.BlockSpec((_TR, din), lambda i: (0, 0)),
            pl.BlockSpec((_TR, dout), lambda i: (0, 0)),
        ],
        out_specs=pl.BlockSpec((_MBLK, dout), lambda i: (i, 0)),
        out_shape=jax.ShapeDtypeStruct((ntok, dout), jnp.float32),
        scratch_shapes=[pltpu.VMEM((dout, din), jnp.bfloat16)],
    )(x, tid2d, W, b2d, a_cat, b_cat)


# speedup vs baseline: 1.0323x; 1.0323x over previous
import jax
import jax.numpy as jnp
from jax.experimental import pallas as pl
from jax.experimental.pallas import tpu as pltpu

_T = 8
_R = 16
_TR = _T * _R
_SCALING = 32.0 / 16.0
_MBLK = 512


def _fused_body(x_ref, tid_ref, w_ref, b_ref, a_ref, bcat_ref, o_ref, wb_ref):
    @pl.when(pl.program_id(0) == 0)
    def _():
        wb_ref[...] = w_ref[...].astype(jnp.bfloat16)

    x = x_ref[...].astype(jnp.bfloat16)
    z = jax.lax.dot_general(
        x, a_ref[...], (((1,), (1,)), ((), ())),
        preferred_element_type=jnp.float32)
    base = jax.lax.dot_general(
        x, wb_ref[...], (((1,), (1,)), ((), ())),
        preferred_element_type=jnp.float32)
    tid = tid_ref[...]
    col = jax.lax.broadcasted_iota(jnp.int32, z.shape, 1)
    onehot = ((col >> 4) == tid).astype(jnp.bfloat16)
    lora = jax.lax.dot_general(
        z.astype(jnp.bfloat16) * onehot, bcat_ref[...], (((1,), (0,)), ((), ())),
        preferred_element_type=jnp.float32)
    o_ref[...] = base + (lora + b_ref[...])


def kernel(x, task_ids, W, b, lora_A, lora_B):
    ntok, din = x.shape
    dout = W.shape[0]
    tid2d = task_ids.astype(jnp.int32).reshape(ntok, 1)
    a_cat = lora_A.reshape(_TR, din).astype(jnp.bfloat16)
    b_cat = jnp.transpose(lora_B, (0, 2, 1)).reshape(_TR, dout)
    b_cat = (b_cat * _SCALING).astype(jnp.bfloat16)
    b2d = b.reshape(1, dout)

    grid = (ntok // _MBLK,)
    return pl.pallas_call(
        _fused_body,
        grid=grid,
        in_specs=[
            pl.BlockSpec((_MBLK, din), lambda i: (i, 0)),
            pl.BlockSpec((_MBLK, 1), lambda i: (i, 0)),
            pl.BlockSpec((dout, din), lambda i: (0, 0)),
            pl.BlockSpec((1, dout), lambda i: (0, 0)),
            pl.BlockSpec((_TR, din), lambda i: (0, 0)),
            pl.BlockSpec((_TR, dout), lambda i: (0, 0)),
        ],
        out_specs=pl.BlockSpec((_MBLK, dout), lambda i: (i, 0)),
        out_shape=jax.ShapeDtypeStruct((ntok, dout), jnp.float32),
        scratch_shapes=[pltpu.VMEM((dout, din), jnp.bfloat16)],
    )(x, tid2d, W, b2d, a_cat, b_cat)


# speedup vs baseline: 1.1579x; 1.1216x over previous
import jax
import jax.numpy as jnp
from jax.experimental import pallas as pl
from jax.experimental.pallas import tpu as pltpu

_T = 8
_R = 16
_TR = _T * _R
_SCALING = 32.0 / 16.0
_MBLK = 512
_KCAT = 2048 + _TR


def _fused_body(x_ref, tid_ref, w_ref, b_ref, a_ref, balt_ref, o_ref,
                rhs_ref, lhs_ref):
    din = w_ref.shape[1]
    @pl.when(pl.program_id(0) == 0)
    def _():
        rhs_ref[:, :din] = w_ref[...].astype(jnp.bfloat16)
        rhs_ref[:, din:] = balt_ref[...]

    xb = x_ref[...].astype(jnp.bfloat16)
    lhs_ref[:, :din] = xb
    z = jax.lax.dot_general(
        xb, a_ref[...], (((1,), (1,)), ((), ())),
        preferred_element_type=jnp.float32)
    tid = tid_ref[...]
    col = jax.lax.broadcasted_iota(jnp.int32, z.shape, 1)
    onehot = ((col >> 4) == tid).astype(jnp.bfloat16)
    lhs_ref[:, din:] = z.astype(jnp.bfloat16) * onehot
    combined = jax.lax.dot_general(
        lhs_ref[...], rhs_ref[...], (((1,), (1,)), ((), ())),
        preferred_element_type=jnp.float32)
    o_ref[...] = combined + b_ref[...]


def kernel(x, task_ids, W, b, lora_A, lora_B):
    ntok, din = x.shape
    dout = W.shape[0]
    tid2d = task_ids.astype(jnp.int32).reshape(ntok, 1)
    a_cat = lora_A.reshape(_TR, din).astype(jnp.bfloat16)
    b_alt = jnp.transpose(lora_B, (1, 0, 2)).reshape(dout, _TR)
    b_alt = (b_alt * _SCALING).astype(jnp.bfloat16)
    b2d = b.reshape(1, dout)

    grid = (ntok // _MBLK,)
    return pl.pallas_call(
        _fused_body,
        grid=grid,
        in_specs=[
            pl.BlockSpec((_MBLK, din), lambda i: (i, 0)),
            pl.BlockSpec((_MBLK, 1), lambda i: (i, 0)),
            pl.BlockSpec((dout, din), lambda i: (0, 0)),
            pl.BlockSpec((1, dout), lambda i: (0, 0)),
            pl.BlockSpec((_TR, din), lambda i: (0, 0)),
            pl.BlockSpec((dout, _TR), lambda i: (0, 0)),
        ],
        out_specs=pl.BlockSpec((_MBLK, dout), lambda i: (i, 0)),
        out_shape=jax.ShapeDtypeStruct((ntok, dout), jnp.float32),
        scratch_shapes=[
            pltpu.VMEM((dout, _KCAT), jnp.bfloat16),
            pltpu.VMEM((_MBLK, _KCAT), jnp.bfloat16),
        ],
    )(x, tid2d, W, b2d, a_cat, b_alt)
